# baseline (speedup 1.0000x reference)
import functools
import jax
import jax.numpy as jnp
from jax import lax
from jax.experimental import pallas as pl
from jax.experimental.pallas import tpu as pltpu
from jax.experimental.pallas import tpu_sc as plsc

H = 8
C = 16

NC = 2
NS = 16
NW = NC * NS


def _proj_body(h_ref, w_ref, bq_ref, bkv_ref, q_ref, kv_ref):
    h = h_ref[...]
    out = jnp.dot(h, w_ref[...], preferred_element_type=jnp.float32)
    q_ref[...] = (out[:, :128] + bq_ref[...]) * 0.25
    kv_ref[...] = out[:, 128:] + bkv_ref[...]


def _tc_proj(h, w, bq, bkv, blk):
    n = h.shape[0]
    grid = (n // blk,)
    return pl.pallas_call(
        _proj_body,
        grid=grid,
        in_specs=[
            pl.BlockSpec((blk, 128), lambda i: (i, 0)),
            pl.BlockSpec((128, 384), lambda i: (0, 0)),
            pl.BlockSpec((1, 128), lambda i: (0, 0)),
            pl.BlockSpec((1, 256), lambda i: (0, 0)),
        ],
        out_specs=[
            pl.BlockSpec((blk, 128), lambda i: (i, 0)),
            pl.BlockSpec((blk, 256), lambda i: (i, 0)),
        ],
        out_shape=[
            jax.ShapeDtypeStruct((n, 128), jnp.float32),
            jax.ShapeDtypeStruct((n, 256), jnp.float32),
        ],
    )(h, w, bq, bkv)


def _in_proj_body(x_ref, w_ref, b_ref, o_ref):
    o_ref[...] = jnp.dot(x_ref[...], w_ref[...],
                         preferred_element_type=jnp.float32) + b_ref[...]


def _tc_in_proj(x, w, b, blk):
    n = x.shape[0]
    return pl.pallas_call(
        _in_proj_body,
        grid=(n // blk,),
        in_specs=[
            pl.BlockSpec((blk, 128), lambda i: (i, 0)),
            pl.BlockSpec((128, 128), lambda i: (0, 0)),
            pl.BlockSpec((1, 128), lambda i: (0, 0)),
        ],
        out_specs=pl.BlockSpec((blk, 128), lambda i: (i, 0)),
        out_shape=jax.ShapeDtypeStruct((n, 128), jnp.float32),
    )(x, w, b)


def _post_body(h_ref, acc_ref, ex_ref, ws_ref, bs_ref, wbo_ref, wbx_ref,
               bb_ref, g_ref, b_ref, o_ref):
    h = h_ref[...]
    msg = acc_ref[0] + acc_ref[1]
    exs = jnp.sum(ex_ref[...], axis=0)
    lane = lax.broadcasted_iota(jnp.int32, (H, 128), 1) // 16
    row = lax.broadcasted_iota(jnp.int32, (H, 128), 0)
    sel = jnp.where(lane == row, 1.0, 0.0).astype(jnp.float32)
    denom = jnp.dot(exs, sel, preferred_element_type=jnp.float32) + 1e-16
    out = msg / denom
    xr = jnp.dot(h, ws_ref[...], preferred_element_type=jnp.float32) + bs_ref[...]
    z = (jnp.dot(out, wbo_ref[...], preferred_element_type=jnp.float32)
         + jnp.dot(xr, wbx_ref[...], preferred_element_type=jnp.float32)
         + bb_ref[...])
    beta = 1.0 / (1.0 + jnp.exp(-z))
    hnew = beta * xr + (1.0 - beta) * out
    y = h + hnew
    mu = jnp.mean(y, axis=1, keepdims=True)
    d = y - mu
    var = jnp.mean(d * d, axis=1, keepdims=True)
    o_ref[...] = d * lax.rsqrt(var + 1e-5) * g_ref[...] + b_ref[...]


def _tc_post(h, acc, ex3, ws, bs, wbo, wbx, bb, g, b, blk):
    n = h.shape[0]
    return pl.pallas_call(
        _post_body,
        grid=(n // blk,),
        in_specs=[
            pl.BlockSpec((blk, 128), lambda i: (i, 0)),
            pl.BlockSpec((2, blk, 128), lambda i: (0, i, 0)),
            pl.BlockSpec((NC, blk, H), lambda i: (0, i, 0)),
            pl.BlockSpec((128, 128), lambda i: (0, 0)),
            pl.BlockSpec((1, 128), lambda i: (0, 0)),
            pl.BlockSpec((128, 1), lambda i: (0, 0)),
            pl.BlockSpec((128, 1), lambda i: (0, 0)),
            pl.BlockSpec((1, 1), lambda i: (0, 0)),
            pl.BlockSpec((1, 128), lambda i: (0, 0)),
            pl.BlockSpec((1, 128), lambda i: (0, 0)),
        ],
        out_specs=pl.BlockSpec((blk, 128), lambda i: (i, 0)),
        out_shape=jax.ShapeDtypeStruct((n, 128), jnp.float32),
    )(h, acc, ex3, ws, bs, wbo, wbx, bb, g, b)


def _pool_body(h_ref, b_ref, loc_ref, glob_ref, cnt_ref):
    i = pl.program_id(0)
    nb = pl.num_programs(0)
    blk = h_ref.shape[0]
    h = h_ref[...]
    bcol = b_ref[...]
    grow = lax.broadcasted_iota(jnp.int32, (1, 32), 1)
    mask = (bcol == grow).astype(jnp.float32)
    dnums = (((0,), (0,)), ((), ()))
    sums = lax.dot_general(mask, h, dnums, preferred_element_type=jnp.float32)
    cnts = lax.dot_general(mask, jnp.ones((blk, 128), jnp.float32), dnums,
                           preferred_element_type=jnp.float32)

    @pl.when(i == 0)
    def _init():
        loc_ref[...] = jnp.zeros_like(loc_ref)
        cnt_ref[...] = jnp.zeros_like(cnt_ref)
        glob_ref[...] = jnp.full_like(glob_ref, -jnp.inf)

    loc_ref[...] += sums
    cnt_ref[...] += cnts
    maskb = bcol == grow
    for g in range(32):
        m = jnp.where(maskb[:, g:g + 1], h, -jnp.inf)
        mx = jnp.max(m, axis=0, keepdims=True)
        glob_ref[g:g + 1, :] = jnp.maximum(glob_ref[g:g + 1, :], mx)

    @pl.when(i == nb - 1)
    def _fin():
        loc_ref[...] = loc_ref[...] / jnp.maximum(cnt_ref[...], 1.0)


def _tc_pool(h, batch2d, blk):
    n = h.shape[0]
    return pl.pallas_call(
        _pool_body,
        grid=(n // blk,),
        in_specs=[
            pl.BlockSpec((blk, 128), lambda i: (i, 0)),
            pl.BlockSpec((blk, 1), lambda i: (i, 0)),
        ],
        out_specs=[
            pl.BlockSpec((32, 128), lambda i: (0, 0)),
            pl.BlockSpec((32, 128), lambda i: (0, 0)),
            pl.BlockSpec((32, 128), lambda i: (0, 0)),
        ],
        out_shape=[
            jax.ShapeDtypeStruct((32, 128), jnp.float32),
            jax.ShapeDtypeStruct((32, 128), jnp.float32),
            jax.ShapeDtypeStruct((32, 128), jnp.float32),
        ],
        compiler_params=pltpu.CompilerParams(
            dimension_semantics=("arbitrary",)),
    )(h, batch2d)


def _make_edge_kernel(n_nodes, n_edges, b_chunk):
    epw = n_edges // NW
    nchunks = epw // b_chunk
    npairs = nchunks // 2
    assert nchunks % 2 == 0 and epw % b_chunk == 0
    npad = -(-n_nodes // (NS * 8)) * (NS * 8)
    rows_per_tile = npad // NS
    nex = npad // 16
    gidx_starts = sorted(set(list(range(0, b_chunk - 15, 16)) + [b_chunk - 16]))
    mesh = plsc.VectorSubcoreMesh(core_axis_name="c", subcore_axis_name="s",
                                  num_cores=NC, num_subcores=NS)

    @functools.partial(
        pl.kernel,
        out_type=[
            jax.ShapeDtypeStruct((NC, npad, 128), jnp.float32),
            jax.ShapeDtypeStruct((NC, nex, 128), jnp.float32),
        ],
        mesh=mesh,
        compiler_params=pltpu.CompilerParams(needs_layout_passes=False),
        scratch_types=[
            pltpu.VMEM((b_chunk,), jnp.int32),
            pltpu.VMEM((b_chunk,), jnp.int32),
            pltpu.VMEM((b_chunk,), jnp.int32),
            pltpu.VMEM((b_chunk,), jnp.int32),
            pltpu.VMEM((2 * b_chunk,), jnp.int32),
            pltpu.VMEM((b_chunk, 128), jnp.float32),
            pltpu.VMEM((b_chunk, 128), jnp.float32),
            pltpu.VMEM((b_chunk, 256), jnp.float32),
            pltpu.VMEM((b_chunk, 256), jnp.float32),
            pltpu.VMEM((2 * b_chunk, 128), jnp.float32),
            pltpu.VMEM_SHARED((npad + nex, 128), jnp.float32),
            pltpu.SemaphoreType.DMA,
            pltpu.SemaphoreType.DMA,
            pltpu.SemaphoreType.DMA,
            pltpu.SemaphoreType.DMA,
            pltpu.SemaphoreType.DMA,
        ],
    )
    def edge_kernel(q_hbm, kv_hbm, src_hbm, dst_hbm, zero_hbm,
                    msg_out, ex_out,
                    srcv0, srcv1, dstv0, dstv1, didx,
                    qv0, qv1, kvv0, kvv1, msgex, accum,
                    isem0, isem1, gsem0, gsem1, ssem):
        cid = lax.axis_index("c")
        sid = lax.axis_index("s")
        wid = sid * NC + cid
        lane_iota = lax.broadcasted_iota(jnp.int32, (16,), 0)
        lane_mask = lane_iota < H
        zvec = jnp.zeros((16,), jnp.float32)
        idx15 = jnp.full((16, 1), 15, jnp.int32)
        gdn = lax.GatherDimensionNumbers(offset_dims=(),
                                         collapsed_slice_dims=(0,),
                                         start_index_map=(0,))

        r0 = sid * rows_per_tile
        pltpu.sync_copy(zero_hbm.at[pl.ds(r0, rows_per_tile)],
                        accum.at[pl.ds(r0, rows_per_tile)])

        @pl.when(sid == 0)
        def _zero_ex():
            pltpu.sync_copy(zero_hbm.at[pl.ds(0, nex)],
                            accum.at[pl.ds(npad, nex)])

        plsc.subcore_barrier()

        def issue_idx(ci, sv, dv, isem):
            base = wid * epw + ci * b_chunk
            c1 = pltpu.async_copy(src_hbm.at[pl.ds(base, b_chunk)], sv, isem)
            c2 = pltpu.async_copy(dst_hbm.at[pl.ds(base, b_chunk)], dv, isem)
            return c1, c2

        def issue_gathers(sv, dv, q, kv, gsem):
            c1 = pltpu.async_copy(q_hbm.at[dv], q, gsem)
            c2 = pltpu.async_copy(kv_hbm.at[sv], kv, gsem)
            return c1, c2

        def compute_chunk(dv, q, kv):
            for s0 in gidx_starts:
                dvg = dv[pl.ds(s0, 16)]
                didx[pl.ds(s0, 16)] = dvg
                didx[pl.ds(b_chunk + s0, 16)] = dvg // 16 + npad

            @plsc.parallel_loop(0, b_chunk, 1, unroll=4)
            def edge_body(e):
                qs = [q[e, pl.ds(hh * 16, 16)] for hh in range(H)]
                ks = [kv[e, pl.ds(hh * 16, 16)] for hh in range(H)]
                vs = [kv[e, pl.ds(128 + hh * 16, 16)] for hh in range(H)]
                exrow = zvec
                exs = []
                for hh in range(H):
                    s = plsc.cumsum(qs[hh] * ks[hh])
                    a = lax.gather(
                        s, idx15, gdn, (1,),
                        mode=lax.GatherScatterMode.PROMISE_IN_BOUNDS)
                    ex = jnp.exp(a)
                    exs.append(ex)
                    exrow = jnp.where(lane_iota == hh, ex, exrow)
                for hh in range(H):
                    msgex[e, pl.ds(hh * 16, 16)] = vs[hh] * exs[hh]
                    msgex[b_chunk + e, pl.ds(hh * 16, 16)] = zvec
                dvec = plsc.load_gather(dv, [jnp.full((16,), e, jnp.int32)])
                col = (dvec % 16) * H + lane_iota
                erow = jnp.full((16,), b_chunk + e, jnp.int32)
                plsc.store_scatter(msgex, [erow, col], exrow, mask=lane_mask)

        def pair_body(p, _):
            a = 2 * p
            i0 = issue_idx(a, srcv0, dstv0, isem0)
            i1 = issue_idx(a + 1, srcv1, dstv1, isem1)
            for c in i0:
                c.wait()
            g0 = issue_gathers(srcv0, dstv0, qv0, kvv0, gsem0)
            for c in i1:
                c.wait()
            g1 = issue_gathers(srcv1, dstv1, qv1, kvv1, gsem1)
            for c in g0:
                c.wait()
            compute_chunk(dstv0, qv0, kvv0)
            sc0 = pltpu.async_copy(msgex, accum.at[didx], ssem, add=True)
            for c in g1:
                c.wait()
            sc0.wait()
            compute_chunk(dstv1, qv1, kvv1)
            pltpu.sync_copy(msgex, accum.at[didx], add=True)
            return 0

        lax.fori_loop(0, npairs, pair_body, 0)
        plsc.subcore_barrier()
        pltpu.sync_copy(accum.at[pl.ds(r0, rows_per_tile)],
                        msg_out.at[cid, pl.ds(r0, rows_per_tile)])

        @pl.when(sid == 0)
        def _copy_ex():
            pltpu.sync_copy(accum.at[pl.ds(npad, nex)], ex_out.at[cid])

    return edge_kernel


def kernel(x, edge_index, batch, W_in, b_in, Wq, bq, Wk, bk, Wv, bv,
           Wskip, bskip, Wbeta, bbeta, ln_g, ln_b):
    n, d = x.shape
    n_edges = edge_index.shape[1]
    n_layers = Wq.shape[0]
    blk = 400 if n % 400 == 0 else n
    epw = n_edges // NW
    b_chunk = 40 if epw % 80 == 0 else epw

    edge_fn = _make_edge_kernel(n, n_edges, b_chunk)
    npad = -(-n // (NS * 8)) * (NS * 8)
    zero = jnp.zeros((npad, 128), jnp.float32)

    src_e = edge_index[0]
    dst_e = edge_index[1]
    h = _tc_in_proj(x, W_in, b_in.reshape(1, d), blk)
    for l in range(n_layers):
        wqkv = jnp.concatenate([Wq[l], Wk[l], Wv[l]], axis=1)
        bkv = jnp.concatenate([bk[l], bv[l]]).reshape(1, 2 * d)
        q, kv = _tc_proj(h, wqkv, bq[l].reshape(1, d), bkv, blk)
        acc, acc_ex = edge_fn(q, kv, src_e, dst_e, zero)
        ex3 = acc_ex.reshape(NC, npad, H)
        wb = Wbeta[l]
        wbo = wb[:d] + wb[2 * d:]
        wbx = wb[d:2 * d] - wb[2 * d:]
        h = _tc_post(h, acc, ex3, Wskip[l], bskip[l].reshape(1, d), wbo, wbx,
                     bbeta[l].reshape(1, 1), ln_g[l].reshape(1, d),
                     ln_b[l].reshape(1, d), blk)

    local_repr, global_repr, _ = _tc_pool(h, batch.reshape(n, 1), blk)
    return local_repr, global_repr, h


# speedup vs baseline: 1.0189x; 1.0189x over previous
import functools
import jax
import jax.numpy as jnp
from jax import lax
from jax.experimental import pallas as pl
from jax.experimental.pallas import tpu as pltpu
from jax.experimental.pallas import tpu_sc as plsc

H = 8
C = 16

NC = 2
NS = 16
NW = NC * NS


def _proj_body(h_ref, w_ref, bq_ref, bkv_ref, q_ref, kv_ref):
    h = h_ref[...]
    out = jnp.dot(h, w_ref[...], preferred_element_type=jnp.float32)
    q_ref[...] = (out[:, :128] + bq_ref[...]) * 0.25
    kv_ref[...] = out[:, 128:] + bkv_ref[...]


def _tc_proj(h, w, bq, bkv, blk):
    n = h.shape[0]
    grid = (n // blk,)
    return pl.pallas_call(
        _proj_body,
        grid=grid,
        in_specs=[
            pl.BlockSpec((blk, 128), lambda i: (i, 0)),
            pl.BlockSpec((128, 384), lambda i: (0, 0)),
            pl.BlockSpec((1, 128), lambda i: (0, 0)),
            pl.BlockSpec((1, 256), lambda i: (0, 0)),
        ],
        out_specs=[
            pl.BlockSpec((blk, 128), lambda i: (i, 0)),
            pl.BlockSpec((blk, 256), lambda i: (i, 0)),
        ],
        out_shape=[
            jax.ShapeDtypeStruct((n, 128), jnp.float32),
            jax.ShapeDtypeStruct((n, 256), jnp.float32),
        ],
    )(h, w, bq, bkv)


def _in_proj_body(x_ref, w_ref, b_ref, wqkv_ref, bq_ref, bkv_ref,
                  h_ref, q_ref, kv_ref):
    h = jnp.dot(x_ref[...], w_ref[...],
                preferred_element_type=jnp.float32) + b_ref[...]
    h_ref[...] = h
    out = jnp.dot(h, wqkv_ref[...], preferred_element_type=jnp.float32)
    q_ref[...] = (out[:, :128] + bq_ref[...]) * 0.25
    kv_ref[...] = out[:, 128:] + bkv_ref[...]


def _tc_in_proj(x, w, b, wqkv, bq, bkv, blk):
    n = x.shape[0]
    return pl.pallas_call(
        _in_proj_body,
        grid=(n // blk,),
        in_specs=[
            pl.BlockSpec((blk, 128), lambda i: (i, 0)),
            pl.BlockSpec((128, 128), lambda i: (0, 0)),
            pl.BlockSpec((1, 128), lambda i: (0, 0)),
            pl.BlockSpec((128, 384), lambda i: (0, 0)),
            pl.BlockSpec((1, 128), lambda i: (0, 0)),
            pl.BlockSpec((1, 256), lambda i: (0, 0)),
        ],
        out_specs=[
            pl.BlockSpec((blk, 128), lambda i: (i, 0)),
            pl.BlockSpec((blk, 128), lambda i: (i, 0)),
            pl.BlockSpec((blk, 256), lambda i: (i, 0)),
        ],
        out_shape=[
            jax.ShapeDtypeStruct((n, 128), jnp.float32),
            jax.ShapeDtypeStruct((n, 128), jnp.float32),
            jax.ShapeDtypeStruct((n, 256), jnp.float32),
        ],
    )(x, w, b, wqkv, bq, bkv)


def _post_body(h_ref, acc_ref, ex_ref, ws_ref, bs_ref, wbo_ref, wbx_ref,
               bb_ref, g_ref, b_ref, o_ref):
    h = h_ref[...]
    msg = acc_ref[0] + acc_ref[1]
    exs = jnp.sum(ex_ref[...], axis=0)
    lane = lax.broadcasted_iota(jnp.int32, (H, 128), 1) // 16
    row = lax.broadcasted_iota(jnp.int32, (H, 128), 0)
    sel = jnp.where(lane == row, 1.0, 0.0).astype(jnp.float32)
    denom = jnp.dot(exs, sel, preferred_element_type=jnp.float32) + 1e-16
    out = msg / denom
    xr = jnp.dot(h, ws_ref[...], preferred_element_type=jnp.float32) + bs_ref[...]
    z = (jnp.dot(out, wbo_ref[...], preferred_element_type=jnp.float32)
         + jnp.dot(xr, wbx_ref[...], preferred_element_type=jnp.float32)
         + bb_ref[...])
    beta = 1.0 / (1.0 + jnp.exp(-z))
    hnew = beta * xr + (1.0 - beta) * out
    y = h + hnew
    mu = jnp.mean(y, axis=1, keepdims=True)
    d = y - mu
    var = jnp.mean(d * d, axis=1, keepdims=True)
    o_ref[...] = d * lax.rsqrt(var + 1e-5) * g_ref[...] + b_ref[...]


def _tc_post(h, acc, ex3, ws, bs, wbo, wbx, bb, g, b, blk):
    n = h.shape[0]
    return pl.pallas_call(
        _post_body,
        grid=(n // blk,),
        in_specs=[
            pl.BlockSpec((blk, 128), lambda i: (i, 0)),
            pl.BlockSpec((2, blk, 128), lambda i: (0, i, 0)),
            pl.BlockSpec((NC, blk, H), lambda i: (0, i, 0)),
            pl.BlockSpec((128, 128), lambda i: (0, 0)),
            pl.BlockSpec((1, 128), lambda i: (0, 0)),
            pl.BlockSpec((128, 1), lambda i: (0, 0)),
            pl.BlockSpec((128, 1), lambda i: (0, 0)),
            pl.BlockSpec((1, 1), lambda i: (0, 0)),
            pl.BlockSpec((1, 128), lambda i: (0, 0)),
            pl.BlockSpec((1, 128), lambda i: (0, 0)),
        ],
        out_specs=pl.BlockSpec((blk, 128), lambda i: (i, 0)),
        out_shape=jax.ShapeDtypeStruct((n, 128), jnp.float32),
    )(h, acc, ex3, ws, bs, wbo, wbx, bb, g, b)


def _post_proj_body(h_ref, acc_ref, ex_ref, ws_ref, bs_ref, wbo_ref,
                    wbx_ref, bb_ref, g_ref, b_ref, wqkv_ref, bq_ref, bkv_ref,
                    o_ref, q_ref, kv_ref):
    h = h_ref[...]
    msg = acc_ref[0] + acc_ref[1]
    exs = jnp.sum(ex_ref[...], axis=0)
    lane = lax.broadcasted_iota(jnp.int32, (H, 128), 1) // 16
    row = lax.broadcasted_iota(jnp.int32, (H, 128), 0)
    sel = jnp.where(lane == row, 1.0, 0.0).astype(jnp.float32)
    denom = jnp.dot(exs, sel, preferred_element_type=jnp.float32) + 1e-16
    out = msg / denom
    xr = jnp.dot(h, ws_ref[...], preferred_element_type=jnp.float32) + bs_ref[...]
    z = (jnp.dot(out, wbo_ref[...], preferred_element_type=jnp.float32)
         + jnp.dot(xr, wbx_ref[...], preferred_element_type=jnp.float32)
         + bb_ref[...])
    beta = 1.0 / (1.0 + jnp.exp(-z))
    hnew = beta * xr + (1.0 - beta) * out
    y = h + hnew
    mu = jnp.mean(y, axis=1, keepdims=True)
    d = y - mu
    var = jnp.mean(d * d, axis=1, keepdims=True)
    hn = d * lax.rsqrt(var + 1e-5) * g_ref[...] + b_ref[...]
    o_ref[...] = hn
    out2 = jnp.dot(hn, wqkv_ref[...], preferred_element_type=jnp.float32)
    q_ref[...] = (out2[:, :128] + bq_ref[...]) * 0.25
    kv_ref[...] = out2[:, 128:] + bkv_ref[...]


def _tc_post_proj(h, acc, ex3, ws, bs, wbo, wbx, bb, g, b,
                  wqkv, bq, bkv, blk):
    n = h.shape[0]
    return pl.pallas_call(
        _post_proj_body,
        grid=(n // blk,),
        in_specs=[
            pl.BlockSpec((blk, 128), lambda i: (i, 0)),
            pl.BlockSpec((2, blk, 128), lambda i: (0, i, 0)),
            pl.BlockSpec((NC, blk, H), lambda i: (0, i, 0)),
            pl.BlockSpec((128, 128), lambda i: (0, 0)),
            pl.BlockSpec((1, 128), lambda i: (0, 0)),
            pl.BlockSpec((128, 1), lambda i: (0, 0)),
            pl.BlockSpec((128, 1), lambda i: (0, 0)),
            pl.BlockSpec((1, 1), lambda i: (0, 0)),
            pl.BlockSpec((1, 128), lambda i: (0, 0)),
            pl.BlockSpec((1, 128), lambda i: (0, 0)),
            pl.BlockSpec((128, 384), lambda i: (0, 0)),
            pl.BlockSpec((1, 128), lambda i: (0, 0)),
            pl.BlockSpec((1, 256), lambda i: (0, 0)),
        ],
        out_specs=[
            pl.BlockSpec((blk, 128), lambda i: (i, 0)),
            pl.BlockSpec((blk, 128), lambda i: (i, 0)),
            pl.BlockSpec((blk, 256), lambda i: (i, 0)),
        ],
        out_shape=[
            jax.ShapeDtypeStruct((n, 128), jnp.float32),
            jax.ShapeDtypeStruct((n, 128), jnp.float32),
            jax.ShapeDtypeStruct((n, 256), jnp.float32),
        ],
    )(h, acc, ex3, ws, bs, wbo, wbx, bb, g, b, wqkv, bq, bkv)


def _pool_body(h_ref, b_ref, loc_ref, glob_ref, cnt_ref):
    i = pl.program_id(0)
    nb = pl.num_programs(0)
    blk = h_ref.shape[0]
    h = h_ref[...]
    bcol = b_ref[...]
    grow = lax.broadcasted_iota(jnp.int32, (1, 32), 1)
    mask = (bcol == grow).astype(jnp.float32)
    dnums = (((0,), (0,)), ((), ()))
    sums = lax.dot_general(mask, h, dnums, preferred_element_type=jnp.float32)
    cnts = lax.dot_general(mask, jnp.ones((blk, 128), jnp.float32), dnums,
                           preferred_element_type=jnp.float32)

    @pl.when(i == 0)
    def _init():
        loc_ref[...] = jnp.zeros_like(loc_ref)
        cnt_ref[...] = jnp.zeros_like(cnt_ref)
        glob_ref[...] = jnp.full_like(glob_ref, -jnp.inf)

    loc_ref[...] += sums
    cnt_ref[...] += cnts
    maskb = bcol == grow
    for g in range(32):
        m = jnp.where(maskb[:, g:g + 1], h, -jnp.inf)
        mx = jnp.max(m, axis=0, keepdims=True)
        glob_ref[g:g + 1, :] = jnp.maximum(glob_ref[g:g + 1, :], mx)

    @pl.when(i == nb - 1)
    def _fin():
        loc_ref[...] = loc_ref[...] / jnp.maximum(cnt_ref[...], 1.0)


def _tc_pool(h, batch2d, blk):
    n = h.shape[0]
    return pl.pallas_call(
        _pool_body,
        grid=(n // blk,),
        in_specs=[
            pl.BlockSpec((blk, 128), lambda i: (i, 0)),
            pl.BlockSpec((blk, 1), lambda i: (i, 0)),
        ],
        out_specs=[
            pl.BlockSpec((32, 128), lambda i: (0, 0)),
            pl.BlockSpec((32, 128), lambda i: (0, 0)),
            pl.BlockSpec((32, 128), lambda i: (0, 0)),
        ],
        out_shape=[
            jax.ShapeDtypeStruct((32, 128), jnp.float32),
            jax.ShapeDtypeStruct((32, 128), jnp.float32),
            jax.ShapeDtypeStruct((32, 128), jnp.float32),
        ],
        compiler_params=pltpu.CompilerParams(
            dimension_semantics=("arbitrary",)),
    )(h, batch2d)


def _make_edge_kernel(n_nodes, n_edges, b_chunk):
    epw = n_edges // NW
    nchunks = epw // b_chunk
    npairs = nchunks // 2
    assert nchunks % 2 == 0 and epw % b_chunk == 0
    npad = -(-n_nodes // (NS * 8)) * (NS * 8)
    rows_per_tile = npad // NS
    nex = npad // 16
    gidx_starts = sorted(set(list(range(0, b_chunk - 15, 16)) + [b_chunk - 16]))
    mesh = plsc.VectorSubcoreMesh(core_axis_name="c", subcore_axis_name="s",
                                  num_cores=NC, num_subcores=NS)

    @functools.partial(
        pl.kernel,
        out_type=[
            jax.ShapeDtypeStruct((NC, npad, 128), jnp.float32),
            jax.ShapeDtypeStruct((NC, nex, 128), jnp.float32),
        ],
        mesh=mesh,
        compiler_params=pltpu.CompilerParams(needs_layout_passes=False),
        scratch_types=[
            pltpu.VMEM((b_chunk,), jnp.int32),
            pltpu.VMEM((b_chunk,), jnp.int32),
            pltpu.VMEM((b_chunk,), jnp.int32),
            pltpu.VMEM((b_chunk,), jnp.int32),
            pltpu.VMEM((2 * b_chunk,), jnp.int32),
            pltpu.VMEM((b_chunk, 128), jnp.float32),
            pltpu.VMEM((b_chunk, 128), jnp.float32),
            pltpu.VMEM((b_chunk, 256), jnp.float32),
            pltpu.VMEM((b_chunk, 256), jnp.float32),
            pltpu.VMEM((2 * b_chunk, 128), jnp.float32),
            pltpu.VMEM_SHARED((npad + nex, 128), jnp.float32),
            pltpu.SemaphoreType.DMA,
            pltpu.SemaphoreType.DMA,
            pltpu.SemaphoreType.DMA,
            pltpu.SemaphoreType.DMA,
            pltpu.SemaphoreType.DMA,
        ],
    )
    def edge_kernel(q_hbm, kv_hbm, src_hbm, dst_hbm, zero_hbm,
                    msg_out, ex_out,
                    srcv0, srcv1, dstv0, dstv1, didx,
                    qv0, qv1, kvv0, kvv1, msgex, accum,
                    isem0, isem1, gsem0, gsem1, ssem):
        cid = lax.axis_index("c")
        sid = lax.axis_index("s")
        wid = sid * NC + cid
        lane_iota = lax.broadcasted_iota(jnp.int32, (16,), 0)
        lane_mask = lane_iota < H
        zvec = jnp.zeros((16,), jnp.float32)
        idx15 = jnp.full((16, 1), 15, jnp.int32)
        gdn = lax.GatherDimensionNumbers(offset_dims=(),
                                         collapsed_slice_dims=(0,),
                                         start_index_map=(0,))

        r0 = sid * rows_per_tile
        pltpu.sync_copy(zero_hbm.at[pl.ds(r0, rows_per_tile)],
                        accum.at[pl.ds(r0, rows_per_tile)])

        @pl.when(sid == 0)
        def _zero_ex():
            pltpu.sync_copy(zero_hbm.at[pl.ds(0, nex)],
                            accum.at[pl.ds(npad, nex)])

        plsc.subcore_barrier()

        def issue_idx(ci, sv, dv, isem):
            base = wid * epw + ci * b_chunk
            c1 = pltpu.async_copy(src_hbm.at[pl.ds(base, b_chunk)], sv, isem)
            c2 = pltpu.async_copy(dst_hbm.at[pl.ds(base, b_chunk)], dv, isem)
            return c1, c2

        def issue_gathers(sv, dv, q, kv, gsem):
            c1 = pltpu.async_copy(q_hbm.at[dv], q, gsem)
            c2 = pltpu.async_copy(kv_hbm.at[sv], kv, gsem)
            return c1, c2

        def compute_chunk(dv, q, kv):
            for s0 in gidx_starts:
                dvg = dv[pl.ds(s0, 16)]
                didx[pl.ds(s0, 16)] = dvg
                didx[pl.ds(b_chunk + s0, 16)] = dvg // 16 + npad

            @plsc.parallel_loop(0, b_chunk, 1, unroll=4)
            def edge_body(e):
                qs = [q[e, pl.ds(hh * 16, 16)] for hh in range(H)]
                ks = [kv[e, pl.ds(hh * 16, 16)] for hh in range(H)]
                vs = [kv[e, pl.ds(128 + hh * 16, 16)] for hh in range(H)]
                exrow = zvec
                exs = []
                for hh in range(H):
                    s = plsc.cumsum(qs[hh] * ks[hh])
                    a = lax.gather(
                        s, idx15, gdn, (1,),
                        mode=lax.GatherScatterMode.PROMISE_IN_BOUNDS)
                    ex = jnp.exp(a)
                    exs.append(ex)
                    exrow = jnp.where(lane_iota == hh, ex, exrow)
                for hh in range(H):
                    msgex[e, pl.ds(hh * 16, 16)] = vs[hh] * exs[hh]
                    msgex[b_chunk + e, pl.ds(hh * 16, 16)] = zvec
                dvec = plsc.load_gather(dv, [jnp.full((16,), e, jnp.int32)])
                col = (dvec % 16) * H + lane_iota
                erow = jnp.full((16,), b_chunk + e, jnp.int32)
                plsc.store_scatter(msgex, [erow, col], exrow, mask=lane_mask)

        def pair_body(p, _):
            a = 2 * p
            i0 = issue_idx(a, srcv0, dstv0, isem0)
            i1 = issue_idx(a + 1, srcv1, dstv1, isem1)
            for c in i0:
                c.wait()
            g0 = issue_gathers(srcv0, dstv0, qv0, kvv0, gsem0)
            for c in i1:
                c.wait()
            g1 = issue_gathers(srcv1, dstv1, qv1, kvv1, gsem1)
            for c in g0:
                c.wait()
            compute_chunk(dstv0, qv0, kvv0)
            sc0 = pltpu.async_copy(msgex, accum.at[didx], ssem, add=True)
            for c in g1:
                c.wait()
            sc0.wait()
            compute_chunk(dstv1, qv1, kvv1)
            pltpu.sync_copy(msgex, accum.at[didx], add=True)
            return 0

        lax.fori_loop(0, npairs, pair_body, 0)
        plsc.subcore_barrier()
        pltpu.sync_copy(accum.at[pl.ds(r0, rows_per_tile)],
                        msg_out.at[cid, pl.ds(r0, rows_per_tile)])

        @pl.when(sid == 0)
        def _copy_ex():
            pltpu.sync_copy(accum.at[pl.ds(npad, nex)], ex_out.at[cid])

    return edge_kernel


def kernel(x, edge_index, batch, W_in, b_in, Wq, bq, Wk, bk, Wv, bv,
           Wskip, bskip, Wbeta, bbeta, ln_g, ln_b):
    n, d = x.shape
    n_edges = edge_index.shape[1]
    n_layers = Wq.shape[0]
    blk = 400 if n % 400 == 0 else n
    epw = n_edges // NW
    b_chunk = 40 if epw % 80 == 0 else epw

    edge_fn = _make_edge_kernel(n, n_edges, b_chunk)
    npad = -(-n // (NS * 8)) * (NS * 8)
    zero = jnp.zeros((npad, 128), jnp.float32)

    src_e = edge_index[0]
    dst_e = edge_index[1]

    def wqkv_of(l):
        return (jnp.concatenate([Wq[l], Wk[l], Wv[l]], axis=1),
                bq[l].reshape(1, d),
                jnp.concatenate([bk[l], bv[l]]).reshape(1, 2 * d))

    w0, bq0, bkv0 = wqkv_of(0)
    h, q, kv = _tc_in_proj(x, W_in, b_in.reshape(1, d), w0, bq0, bkv0, blk)
    for l in range(n_layers):
        acc, acc_ex = edge_fn(q, kv, src_e, dst_e, zero)
        ex3 = acc_ex.reshape(NC, npad, H)
        wb = Wbeta[l]
        wbo = wb[:d] + wb[2 * d:]
        wbx = wb[d:2 * d] - wb[2 * d:]
        args = (h, acc, ex3, Wskip[l], bskip[l].reshape(1, d), wbo, wbx,
                bbeta[l].reshape(1, 1), ln_g[l].reshape(1, d),
                ln_b[l].reshape(1, d))
        if l + 1 < n_layers:
            wn, bqn, bkvn = wqkv_of(l + 1)
            h, q, kv = _tc_post_proj(*args, wn, bqn, bkvn, blk)
        else:
            h = _tc_post(*args, blk)

    local_repr, global_repr, _ = _tc_pool(h, batch.reshape(n, 1), blk)
    return local_repr, global_repr, h


# speedup vs baseline: 1.0211x; 1.0022x over previous
import functools
import jax
import jax.numpy as jnp
from jax import lax
from jax.experimental import pallas as pl
from jax.experimental.pallas import tpu as pltpu
from jax.experimental.pallas import tpu_sc as plsc

H = 8
C = 16

NC = 2
NS = 16
NW = NC * NS


def _in_proj_body(x_ref, w_ref, b_ref, wqkv_ref, bq_ref, bkv_ref,
                  h_ref, q_ref, kv_ref):
    h = jnp.dot(x_ref[...], w_ref[...],
                preferred_element_type=jnp.float32) + b_ref[...]
    h_ref[...] = h
    out = jnp.dot(h, wqkv_ref[...], preferred_element_type=jnp.float32)
    q_ref[...] = (out[:, :128] + bq_ref[...]) * 0.25
    kv_ref[...] = out[:, 128:] + bkv_ref[...]


def _tc_in_proj(x, w, b, wqkv, bq, bkv, blk):
    n = x.shape[0]
    return pl.pallas_call(
        _in_proj_body,
        grid=(n // blk,),
        in_specs=[
            pl.BlockSpec((blk, 128), lambda i: (i, 0)),
            pl.BlockSpec((128, 128), lambda i: (0, 0)),
            pl.BlockSpec((1, 128), lambda i: (0, 0)),
            pl.BlockSpec((128, 384), lambda i: (0, 0)),
            pl.BlockSpec((1, 128), lambda i: (0, 0)),
            pl.BlockSpec((1, 256), lambda i: (0, 0)),
        ],
        out_specs=[
            pl.BlockSpec((blk, 128), lambda i: (i, 0)),
            pl.BlockSpec((blk, 128), lambda i: (i, 0)),
            pl.BlockSpec((blk, 256), lambda i: (i, 0)),
        ],
        out_shape=[
            jax.ShapeDtypeStruct((n, 128), jnp.float32),
            jax.ShapeDtypeStruct((n, 128), jnp.float32),
            jax.ShapeDtypeStruct((n, 256), jnp.float32),
        ],
    )(x, w, b, wqkv, bq, bkv)


def _post_body(h_ref, acc_ref, ex_ref, ws_ref, bs_ref, wbo_ref, wbx_ref,
               bb_ref, g_ref, b_ref, o_ref):
    h = h_ref[...]
    msg = acc_ref[0] + acc_ref[1]
    exs = jnp.sum(ex_ref[...], axis=0)
    lane = lax.broadcasted_iota(jnp.int32, (H, 128), 1) // 16
    row = lax.broadcasted_iota(jnp.int32, (H, 128), 0)
    sel = jnp.where(lane == row, 1.0, 0.0).astype(jnp.float32)
    denom = jnp.dot(exs, sel, preferred_element_type=jnp.float32) + 1e-16
    out = msg / denom
    xr = jnp.dot(h, ws_ref[...], preferred_element_type=jnp.float32) + bs_ref[...]
    z = (jnp.dot(out, wbo_ref[...], preferred_element_type=jnp.float32)
         + jnp.dot(xr, wbx_ref[...], preferred_element_type=jnp.float32)
         + bb_ref[...])
    beta = 1.0 / (1.0 + jnp.exp(-z))
    hnew = beta * xr + (1.0 - beta) * out
    y = h + hnew
    mu = jnp.mean(y, axis=1, keepdims=True)
    d = y - mu
    var = jnp.mean(d * d, axis=1, keepdims=True)
    o_ref[...] = d * lax.rsqrt(var + 1e-5) * g_ref[...] + b_ref[...]


def _tc_post(h, acc, ex3, ws, bs, wbo, wbx, bb, g, b, blk):
    n = h.shape[0]
    return pl.pallas_call(
        _post_body,
        grid=(n // blk,),
        in_specs=[
            pl.BlockSpec((blk, 128), lambda i: (i, 0)),
            pl.BlockSpec((2, blk, 128), lambda i: (0, i, 0)),
            pl.BlockSpec((NC, blk, H), lambda i: (0, i, 0)),
            pl.BlockSpec((128, 128), lambda i: (0, 0)),
            pl.BlockSpec((1, 128), lambda i: (0, 0)),
            pl.BlockSpec((128, 1), lambda i: (0, 0)),
            pl.BlockSpec((128, 1), lambda i: (0, 0)),
            pl.BlockSpec((1, 1), lambda i: (0, 0)),
            pl.BlockSpec((1, 128), lambda i: (0, 0)),
            pl.BlockSpec((1, 128), lambda i: (0, 0)),
        ],
        out_specs=pl.BlockSpec((blk, 128), lambda i: (i, 0)),
        out_shape=jax.ShapeDtypeStruct((n, 128), jnp.float32),
    )(h, acc, ex3, ws, bs, wbo, wbx, bb, g, b)


def _post_proj_body(h_ref, acc_ref, ex_ref, ws_ref, bs_ref, wbo_ref,
                    wbx_ref, bb_ref, g_ref, b_ref, wqkv_ref, bq_ref, bkv_ref,
                    o_ref, q_ref, kv_ref):
    h = h_ref[...]
    msg = acc_ref[0] + acc_ref[1]
    exs = jnp.sum(ex_ref[...], axis=0)
    lane = lax.broadcasted_iota(jnp.int32, (H, 128), 1) // 16
    row = lax.broadcasted_iota(jnp.int32, (H, 128), 0)
    sel = jnp.where(lane == row, 1.0, 0.0).astype(jnp.float32)
    denom = jnp.dot(exs, sel, preferred_element_type=jnp.float32) + 1e-16
    out = msg / denom
    xr = jnp.dot(h, ws_ref[...], preferred_element_type=jnp.float32) + bs_ref[...]
    z = (jnp.dot(out, wbo_ref[...], preferred_element_type=jnp.float32)
         + jnp.dot(xr, wbx_ref[...], preferred_element_type=jnp.float32)
         + bb_ref[...])
    beta = 1.0 / (1.0 + jnp.exp(-z))
    hnew = beta * xr + (1.0 - beta) * out
    y = h + hnew
    mu = jnp.mean(y, axis=1, keepdims=True)
    d = y - mu
    var = jnp.mean(d * d, axis=1, keepdims=True)
    hn = d * lax.rsqrt(var + 1e-5) * g_ref[...] + b_ref[...]
    o_ref[...] = hn
    out2 = jnp.dot(hn, wqkv_ref[...], preferred_element_type=jnp.float32)
    q_ref[...] = (out2[:, :128] + bq_ref[...]) * 0.25
    kv_ref[...] = out2[:, 128:] + bkv_ref[...]


def _tc_post_proj(h, acc, ex3, ws, bs, wbo, wbx, bb, g, b,
                  wqkv, bq, bkv, blk):
    n = h.shape[0]
    return pl.pallas_call(
        _post_proj_body,
        grid=(n // blk,),
        in_specs=[
            pl.BlockSpec((blk, 128), lambda i: (i, 0)),
            pl.BlockSpec((2, blk, 128), lambda i: (0, i, 0)),
            pl.BlockSpec((NC, blk, H), lambda i: (0, i, 0)),
            pl.BlockSpec((128, 128), lambda i: (0, 0)),
            pl.BlockSpec((1, 128), lambda i: (0, 0)),
            pl.BlockSpec((128, 1), lambda i: (0, 0)),
            pl.BlockSpec((128, 1), lambda i: (0, 0)),
            pl.BlockSpec((1, 1), lambda i: (0, 0)),
            pl.BlockSpec((1, 128), lambda i: (0, 0)),
            pl.BlockSpec((1, 128), lambda i: (0, 0)),
            pl.BlockSpec((128, 384), lambda i: (0, 0)),
            pl.BlockSpec((1, 128), lambda i: (0, 0)),
            pl.BlockSpec((1, 256), lambda i: (0, 0)),
        ],
        out_specs=[
            pl.BlockSpec((blk, 128), lambda i: (i, 0)),
            pl.BlockSpec((blk, 128), lambda i: (i, 0)),
            pl.BlockSpec((blk, 256), lambda i: (i, 0)),
        ],
        out_shape=[
            jax.ShapeDtypeStruct((n, 128), jnp.float32),
            jax.ShapeDtypeStruct((n, 128), jnp.float32),
            jax.ShapeDtypeStruct((n, 256), jnp.float32),
        ],
    )(h, acc, ex3, ws, bs, wbo, wbx, bb, g, b, wqkv, bq, bkv)


def _pool_body(h_ref, b_ref, loc_ref, glob_ref, cnt_ref):
    i = pl.program_id(0)
    nb = pl.num_programs(0)
    blk = h_ref.shape[0]
    h = h_ref[...]
    bcol = b_ref[...]
    grow = lax.broadcasted_iota(jnp.int32, (1, 32), 1)
    mask = (bcol == grow).astype(jnp.float32)
    dnums = (((0,), (0,)), ((), ()))
    sums = lax.dot_general(mask, h, dnums, preferred_element_type=jnp.float32)
    cnts = lax.dot_general(mask, jnp.ones((blk, 128), jnp.float32), dnums,
                           preferred_element_type=jnp.float32)

    @pl.when(i == 0)
    def _init():
        loc_ref[...] = jnp.zeros_like(loc_ref)
        cnt_ref[...] = jnp.zeros_like(cnt_ref)
        glob_ref[...] = jnp.full_like(glob_ref, -jnp.inf)

    loc_ref[...] += sums
    cnt_ref[...] += cnts
    maskb = bcol == grow
    for g in range(32):
        m = jnp.where(maskb[:, g:g + 1], h, -jnp.inf)
        mx = jnp.max(m, axis=0, keepdims=True)
        glob_ref[g:g + 1, :] = jnp.maximum(glob_ref[g:g + 1, :], mx)

    @pl.when(i == nb - 1)
    def _fin():
        loc_ref[...] = loc_ref[...] / jnp.maximum(cnt_ref[...], 1.0)


def _tc_pool(h, batch2d, blk):
    n = h.shape[0]
    return pl.pallas_call(
        _pool_body,
        grid=(n // blk,),
        in_specs=[
            pl.BlockSpec((blk, 128), lambda i: (i, 0)),
            pl.BlockSpec((blk, 1), lambda i: (i, 0)),
        ],
        out_specs=[
            pl.BlockSpec((32, 128), lambda i: (0, 0)),
            pl.BlockSpec((32, 128), lambda i: (0, 0)),
            pl.BlockSpec((32, 128), lambda i: (0, 0)),
        ],
        out_shape=[
            jax.ShapeDtypeStruct((32, 128), jnp.float32),
            jax.ShapeDtypeStruct((32, 128), jnp.float32),
            jax.ShapeDtypeStruct((32, 128), jnp.float32),
        ],
        compiler_params=pltpu.CompilerParams(
            dimension_semantics=("arbitrary",)),
    )(h, batch2d)


def _make_edge_kernel(n_nodes, n_edges, b_chunk):
    epw = n_edges // NW
    nchunks = epw // b_chunk
    npairs = nchunks // 2
    assert nchunks % 2 == 0 and epw % b_chunk == 0
    npad = -(-n_nodes // (NS * 8)) * (NS * 8)
    rows_per_tile = npad // NS
    nex = npad // 16
    gidx_starts = sorted(set(list(range(0, b_chunk - 15, 16)) + [b_chunk - 16]))
    mesh = plsc.VectorSubcoreMesh(core_axis_name="c", subcore_axis_name="s",
                                  num_cores=NC, num_subcores=NS)

    @functools.partial(
        pl.kernel,
        out_type=[
            jax.ShapeDtypeStruct((NC, npad, 128), jnp.float32),
            jax.ShapeDtypeStruct((NC, nex, 128), jnp.float32),
        ],
        mesh=mesh,
        compiler_params=pltpu.CompilerParams(needs_layout_passes=False),
        scratch_types=[
            pltpu.VMEM((b_chunk,), jnp.int32),
            pltpu.VMEM((b_chunk,), jnp.int32),
            pltpu.VMEM((b_chunk,), jnp.int32),
            pltpu.VMEM((b_chunk,), jnp.int32),
            pltpu.VMEM((2 * b_chunk,), jnp.int32),
            pltpu.VMEM((b_chunk, 128), jnp.float32),
            pltpu.VMEM((b_chunk, 128), jnp.float32),
            pltpu.VMEM((b_chunk, 256), jnp.float32),
            pltpu.VMEM((b_chunk, 256), jnp.float32),
            pltpu.VMEM((2 * b_chunk, 128), jnp.float32),
            pltpu.VMEM_SHARED((npad + nex, 128), jnp.float32),
            pltpu.SemaphoreType.DMA,
            pltpu.SemaphoreType.DMA,
            pltpu.SemaphoreType.DMA,
            pltpu.SemaphoreType.DMA,
            pltpu.SemaphoreType.DMA,
        ],
    )
    def edge_kernel(q_hbm, kv_hbm, src_hbm, dst_hbm, zero_hbm,
                    msg_out, ex_out,
                    srcv0, srcv1, dstv0, dstv1, didx,
                    qv0, qv1, kvv0, kvv1, msgex, accum,
                    isem0, isem1, gsem0, gsem1, ssem):
        cid = lax.axis_index("c")
        sid = lax.axis_index("s")
        wid = sid * NC + cid
        lane_iota = lax.broadcasted_iota(jnp.int32, (16,), 0)
        lane_mask = lane_iota < H
        zvec = jnp.zeros((16,), jnp.float32)
        idx15 = jnp.full((16, 1), 15, jnp.int32)
        gdn = lax.GatherDimensionNumbers(offset_dims=(),
                                         collapsed_slice_dims=(0,),
                                         start_index_map=(0,))

        r0 = sid * rows_per_tile
        pltpu.sync_copy(zero_hbm.at[pl.ds(r0, rows_per_tile)],
                        accum.at[pl.ds(r0, rows_per_tile)])

        @pl.when(sid == 0)
        def _zero_ex():
            pltpu.sync_copy(zero_hbm.at[pl.ds(0, nex)],
                            accum.at[pl.ds(npad, nex)])

        plsc.subcore_barrier()

        def issue_idx(ci, sv, dv, isem):
            base = wid * epw + ci * b_chunk
            c1 = pltpu.async_copy(src_hbm.at[pl.ds(base, b_chunk)], sv, isem)
            c2 = pltpu.async_copy(dst_hbm.at[pl.ds(base, b_chunk)], dv, isem)
            return c1, c2

        def issue_gathers(sv, dv, q, kv, gsem):
            c1 = pltpu.async_copy(q_hbm.at[dv], q, gsem)
            c2 = pltpu.async_copy(kv_hbm.at[sv], kv, gsem)
            return c1, c2

        def compute_chunk(dv, q, kv):
            for s0 in gidx_starts:
                dvg = dv[pl.ds(s0, 16)]
                didx[pl.ds(s0, 16)] = dvg
                didx[pl.ds(b_chunk + s0, 16)] = dvg // 16 + npad

            @plsc.parallel_loop(0, b_chunk, 1, unroll=4)
            def edge_body(e):
                qs = [q[e, pl.ds(hh * 16, 16)] for hh in range(H)]
                ks = [kv[e, pl.ds(hh * 16, 16)] for hh in range(H)]
                vs = [kv[e, pl.ds(128 + hh * 16, 16)] for hh in range(H)]
                exrow = zvec
                exs = []
                for hh in range(H):
                    s = plsc.cumsum(qs[hh] * ks[hh])
                    a = lax.gather(
                        s, idx15, gdn, (1,),
                        mode=lax.GatherScatterMode.PROMISE_IN_BOUNDS)
                    ex = jnp.exp(a)
                    exs.append(ex)
                    exrow = jnp.where(lane_iota == hh, ex, exrow)
                for hh in range(H):
                    msgex[e, pl.ds(hh * 16, 16)] = vs[hh] * exs[hh]
                    msgex[b_chunk + e, pl.ds(hh * 16, 16)] = zvec
                dvec = plsc.load_gather(dv, [jnp.full((16,), e, jnp.int32)])
                col = (dvec % 16) * H + lane_iota
                erow = jnp.full((16,), b_chunk + e, jnp.int32)
                plsc.store_scatter(msgex, [erow, col], exrow, mask=lane_mask)

        def pair_body(p, _):
            a = 2 * p
            i0 = issue_idx(a, srcv0, dstv0, isem0)
            i1 = issue_idx(a + 1, srcv1, dstv1, isem1)
            for c in i0:
                c.wait()
            g0 = issue_gathers(srcv0, dstv0, qv0, kvv0, gsem0)
            for c in i1:
                c.wait()
            g1 = issue_gathers(srcv1, dstv1, qv1, kvv1, gsem1)
            for c in g0:
                c.wait()
            compute_chunk(dstv0, qv0, kvv0)
            sc0 = pltpu.async_copy(msgex, accum.at[didx], ssem, add=True)
            for c in g1:
                c.wait()
            sc0.wait()
            compute_chunk(dstv1, qv1, kvv1)
            pltpu.sync_copy(msgex, accum.at[didx], add=True)
            return 0

        lax.fori_loop(0, npairs, pair_body, 0)
        plsc.subcore_barrier()
        pltpu.sync_copy(accum.at[pl.ds(r0, rows_per_tile)],
                        msg_out.at[cid, pl.ds(r0, rows_per_tile)])

        @pl.when(sid == 0)
        def _copy_ex():
            pltpu.sync_copy(accum.at[pl.ds(npad, nex)], ex_out.at[cid])

    return edge_kernel


def kernel(x, edge_index, batch, W_in, b_in, Wq, bq, Wk, bk, Wv, bv,
           Wskip, bskip, Wbeta, bbeta, ln_g, ln_b):
    n, d = x.shape
    n_edges = edge_index.shape[1]
    n_layers = Wq.shape[0]
    blk = 400 if n % 400 == 0 else n
    epw = n_edges // NW
    b_chunk = 40 if epw % 80 == 0 else epw

    edge_fn = _make_edge_kernel(n, n_edges, b_chunk)
    npad = -(-n // (NS * 8)) * (NS * 8)
    zero = jnp.zeros((npad, 128), jnp.float32)

    src_e = edge_index[0]
    dst_e = edge_index[1]

    def wqkv_of(l):
        return (jnp.concatenate([Wq[l], Wk[l], Wv[l]], axis=1),
                bq[l].reshape(1, d),
                jnp.concatenate([bk[l], bv[l]]).reshape(1, 2 * d))

    w0, bq0, bkv0 = wqkv_of(0)
    h, q, kv = _tc_in_proj(x, W_in, b_in.reshape(1, d), w0, bq0, bkv0, blk)
    for l in range(n_layers):
        acc, acc_ex = edge_fn(q, kv, src_e, dst_e, zero)
        ex3 = acc_ex.reshape(NC, npad, H)
        wb = Wbeta[l]
        wbo = wb[:d] + wb[2 * d:]
        wbx = wb[d:2 * d] - wb[2 * d:]
        args = (h, acc, ex3, Wskip[l], bskip[l].reshape(1, d), wbo, wbx,
                bbeta[l].reshape(1, 1), ln_g[l].reshape(1, d),
                ln_b[l].reshape(1, d))
        if l + 1 < n_layers:
            wn, bqn, bkvn = wqkv_of(l + 1)
            h, q, kv = _tc_post_proj(*args, wn, bqn, bkvn, blk)
        else:
            h = _tc_post(*args, blk)

    local_repr, global_repr, _ = _tc_pool(h, batch.reshape(n, 1), blk)
    return local_repr, global_repr, h
